# baseline (speedup 1.0000x reference)
import numpy as np
import jax
import jax.numpy as jnp
from jax.experimental import pallas as pl
from jax.experimental.pallas import tpu as pltpu

H = W = 32
C_IN, C1, C2 = 3, 32, 64
C0P = 8
K1P = 9 * C0P
K2 = 9 * C1
WROW = 36
IMGW = H * WROW
HALO = 64
XS = IMGW + 2 * HALO
IPC = 4
CW = IPC * IMGW
POOL_HW = 256
FC1_IN, FC1_OUT, FC2_OUT = C2 * POOL_HW, 128, 10
BM_CONV = 64
BM_MLP = 256
VMEM_LIMIT = 48 * 1024 * 1024

_MASK_NP = np.tile(np.tile(np.concatenate([np.ones(W), np.zeros(WROW - W)]), H),
                   IPC)
_G_NP = np.zeros((IMGW, 2 * POOL_HW), np.float32)
for _q in range(2):
    for _p in range(POOL_HW):
        _G_NP[2 * WROW * (_p // 16) + 2 * (_p % 16) + WROW * _q,
              _q * POOL_HW + _p] = 1.0

_TAP_SHIFTS = [(dy - 1) * WROW + (dx - 1) for dy in range(3) for dx in range(3)]


def _conv_stack_kernel(x_ref, w1_ref, b1_ref, w2_ref, b2_ref, m_ref, g_ref,
                       o_ref, p1t, a1f, p2t, a2all):
    bm = x_ref.shape[0]
    nch = bm // IPC
    mask = m_ref[...]
    a1f[:, 0:HALO] = jnp.zeros((C1, HALO), jnp.bfloat16)
    a1f[:, bm * XS - HALO:bm * XS] = jnp.zeros((C1, HALO), jnp.bfloat16)
    for i in range(bm - 1):
        a1f[:, i * XS + HALO + IMGW:(i + 1) * XS + HALO] = (
            jnp.zeros((C1, 2 * HALO), jnp.bfloat16))

    for c in range(nch):
        for k in range(IPC):
            i = c * IPC + k
            for g, sh in enumerate(_TAP_SHIFTS):
                p1t[C0P * g:C0P * (g + 1), k * IMGW:(k + 1) * IMGW] = (
                    x_ref[i, :, HALO + sh:HALO + sh + IMGW])
        a1 = jnp.dot(w1_ref[...], p1t[...],
                     preferred_element_type=jnp.float32)
        a1 = jnp.maximum(a1 + b1_ref[...], 0.0)
        a1 = (a1 * mask).astype(jnp.bfloat16)
        for k in range(IPC):
            i = c * IPC + k
            a1f[:, i * XS + HALO:i * XS + HALO + IMGW] = (
                a1[:, k * IMGW:(k + 1) * IMGW])

    for c in range(nch):
        for k in range(IPC):
            i = c * IPC + k
            for g, sh in enumerate(_TAP_SHIFTS):
                p2t[C1 * g:C1 * (g + 1), k * IMGW:(k + 1) * IMGW] = (
                    a1f[:, i * XS + HALO + sh:i * XS + HALO + sh + IMGW])
        a2 = jnp.dot(w2_ref[...], p2t[...],
                     preferred_element_type=jnp.float32)
        a2 = jnp.maximum(a2 + b2_ref[...], 0.0).astype(jnp.bfloat16)
        m1 = jnp.maximum(a2, jnp.roll(a2, -1, axis=1))
        for k in range(IPC):
            i = c * IPC + k
            a2all[C2 * i:C2 * (i + 1), :] = m1[:, k * IMGW:(k + 1) * IMGW]

    for c in range(bm * C2 // 512):
        y2 = jnp.dot(a2all[512 * c:512 * (c + 1), :], g_ref[...],
                     preferred_element_type=jnp.float32)
        pooled = jnp.maximum(y2[:, 0:POOL_HW], y2[:, POOL_HW:])
        o_ref[(512 // C2) * c:(512 // C2) * (c + 1)] = (
            pooled.astype(jnp.bfloat16).reshape(512 // C2, C2, POOL_HW))


def _conv_stack(xp, w1t, b1t, w2t, b2t, mask, gsel):
    bsz = xp.shape[0]
    bm = BM_CONV
    while bsz % bm or bm < 8:
        bm //= 2
        if bm < 8:
            raise ValueError("batch must be a multiple of 8")
    return pl.pallas_call(
        _conv_stack_kernel,
        out_shape=jax.ShapeDtypeStruct((bsz, C2, POOL_HW), jnp.bfloat16),
        grid=(bsz // bm,),
        in_specs=[
            pl.BlockSpec((bm, C0P, XS), lambda i: (i, 0, 0)),
            pl.BlockSpec((C1, K1P), lambda i: (0, 0)),
            pl.BlockSpec((C1, 1), lambda i: (0, 0)),
            pl.BlockSpec((C2, K2), lambda i: (0, 0)),
            pl.BlockSpec((C2, 1), lambda i: (0, 0)),
            pl.BlockSpec((1, CW), lambda i: (0, 0)),
            pl.BlockSpec((IMGW, 2 * POOL_HW), lambda i: (0, 0)),
        ],
        out_specs=pl.BlockSpec((bm, C2, POOL_HW), lambda i: (i, 0, 0)),
        scratch_shapes=[
            pltpu.VMEM((K1P, CW), jnp.bfloat16),
            pltpu.VMEM((C1, BM_CONV * XS), jnp.bfloat16),
            pltpu.VMEM((K2, CW), jnp.bfloat16),
            pltpu.VMEM((C2 * BM_CONV, IMGW), jnp.bfloat16),
        ],
        compiler_params=pltpu.CompilerParams(
            dimension_semantics=("parallel",),
            vmem_limit_bytes=VMEM_LIMIT,
        ),
    )(xp, w1t, b1t, w2t, b2t, mask, gsel)


def _mlp_kernel(x_ref, w1_ref, b1_ref, w2_ref, b2_ref, o_ref):
    h = jnp.dot(x_ref[...], w1_ref[...], preferred_element_type=jnp.float32)
    h = jnp.maximum(h + b1_ref[...], 0.0).astype(jnp.bfloat16)
    o_ref[...] = jnp.dot(h, w2_ref[...], preferred_element_type=jnp.float32) + b2_ref[...]


def _mlp(flat, fw1, fb1, fw2, fb2):
    bsz = flat.shape[0]
    bm = BM_MLP if bsz % BM_MLP == 0 else bsz
    return pl.pallas_call(
        _mlp_kernel,
        out_shape=jax.ShapeDtypeStruct((bsz, FC2_OUT), jnp.float32),
        grid=(bsz // bm,),
        in_specs=[
            pl.BlockSpec((bm, FC1_IN), lambda i: (i, 0)),
            pl.BlockSpec((FC1_IN, FC1_OUT), lambda i: (0, 0)),
            pl.BlockSpec((1, FC1_OUT), lambda i: (0, 0)),
            pl.BlockSpec((FC1_OUT, FC2_OUT), lambda i: (0, 0)),
            pl.BlockSpec((1, FC2_OUT), lambda i: (0, 0)),
        ],
        out_specs=pl.BlockSpec((bm, FC2_OUT), lambda i: (i, 0)),
        compiler_params=pltpu.CompilerParams(
            dimension_semantics=("parallel",),
            vmem_limit_bytes=VMEM_LIMIT,
        ),
    )(flat, fw1, fb1, fw2, fb2)


def kernel(x, w1k, b1, w2k, b2, fw1, fb1, fw2, fb2):
    bsz = x.shape[0]
    xb = x.astype(jnp.bfloat16)
    xr = jnp.pad(xb, ((0, 0), (0, C0P - C_IN), (0, 0), (0, WROW - W)))
    xp = jnp.pad(xr.reshape(bsz, C0P, IMGW), ((0, 0), (0, 0), (HALO, HALO)))
    w1t = jnp.pad(w1k.reshape(9, C_IN, C1),
                  ((0, 0), (0, C0P - C_IN), (0, 0))).reshape(K1P, C1)
    w1t = w1t.T.astype(jnp.bfloat16)
    w2t = w2k.T.astype(jnp.bfloat16)
    mask = jnp.asarray(_MASK_NP, jnp.bfloat16)[None, :]
    gsel = jnp.asarray(_G_NP, jnp.bfloat16)
    pooled = _conv_stack(xp, w1t, b1.T, w2t, b2.T, mask, gsel)
    flat = pooled.reshape(bsz, FC1_IN)
    return _mlp(flat, fw1.astype(jnp.bfloat16), fb1,
                fw2.astype(jnp.bfloat16), fb2)


# speedup vs baseline: 1.0100x; 1.0100x over previous
import numpy as np
import jax
import jax.numpy as jnp
from jax.experimental import pallas as pl
from jax.experimental.pallas import tpu as pltpu

H = W = 32
C_IN, C1, C2 = 3, 32, 64
C0P = 8
K1P = 9 * C0P
K2 = 9 * C1
WROW = 36
IMGW = H * WROW
HALO = 64
XS = IMGW + 2 * HALO
IPC = 4
CW = IPC * IMGW
POOL_HW = 256
FC1_IN, FC1_OUT, FC2_OUT = C2 * POOL_HW, 128, 10
BM_CONV = 64
BM_MLP = 256
VMEM_LIMIT = 48 * 1024 * 1024

_MASK_NP = np.tile(np.tile(np.concatenate([np.ones(W), np.zeros(WROW - W)]), H),
                   IPC)
_G_NP = np.zeros((IMGW, 2 * POOL_HW), np.float32)
for _q in range(2):
    for _p in range(POOL_HW):
        _G_NP[2 * WROW * (_p // 16) + 2 * (_p % 16) + WROW * _q,
              _q * POOL_HW + _p] = 1.0

_TAP_SHIFTS = [(dy - 1) * WROW + (dx - 1) for dy in range(3) for dx in range(3)]


def _conv_stack_kernel(x_ref, w1_ref, b1_ref, w2_ref, b2_ref, m_ref, g_ref,
                       o_ref, p1t, a1f, p2t, a2all):
    bm = x_ref.shape[0]
    nch = bm // IPC
    mask = m_ref[...]
    a1f[:, 0:HALO] = jnp.zeros((C1, HALO), jnp.bfloat16)
    a1f[:, bm * XS - HALO:bm * XS] = jnp.zeros((C1, HALO), jnp.bfloat16)
    for i in range(bm - 1):
        a1f[:, i * XS + HALO + IMGW:(i + 1) * XS + HALO] = (
            jnp.zeros((C1, 2 * HALO), jnp.bfloat16))

    for c in range(nch):
        for k in range(IPC):
            i = c * IPC + k
            for g, sh in enumerate(_TAP_SHIFTS):
                p1t[C0P * g:C0P * (g + 1), k * IMGW:(k + 1) * IMGW] = (
                    x_ref[i, :, HALO + sh:HALO + sh + IMGW])
        a1 = jnp.dot(w1_ref[...], p1t[...],
                     preferred_element_type=jnp.float32)
        a1 = jnp.maximum(a1 + b1_ref[...], 0.0)
        a1 = (a1 * mask).astype(jnp.bfloat16)
        for k in range(IPC):
            i = c * IPC + k
            a1f[:, i * XS + HALO:i * XS + HALO + IMGW] = (
                a1[:, k * IMGW:(k + 1) * IMGW])

    for c in range(nch):
        for k in range(IPC):
            i = c * IPC + k
            for g, sh in enumerate(_TAP_SHIFTS):
                p2t[C1 * g:C1 * (g + 1), k * IMGW:(k + 1) * IMGW] = (
                    a1f[:, i * XS + HALO + sh:i * XS + HALO + sh + IMGW])
        a2 = jnp.dot(w2_ref[...], p2t[...],
                     preferred_element_type=jnp.float32)
        a2 = jnp.maximum(a2 + b2_ref[...], 0.0).astype(jnp.bfloat16)
        m1 = jnp.maximum(a2, jnp.roll(a2, -1, axis=1))
        for k in range(IPC):
            i = c * IPC + k
            a2all[C2 * i:C2 * (i + 1), :] = m1[:, k * IMGW:(k + 1) * IMGW]

    for c in range(bm * C2 // 512):
        y2 = jnp.dot(a2all[512 * c:512 * (c + 1), :], g_ref[...],
                     preferred_element_type=jnp.float32)
        pooled = jnp.maximum(y2[:, 0:POOL_HW], y2[:, POOL_HW:])
        o_ref[(512 // C2) * c:(512 // C2) * (c + 1)] = (
            pooled.astype(jnp.bfloat16).reshape(512 // C2, C2, POOL_HW))


def _conv_stack(xp, w1t, b1t, w2t, b2t, mask, gsel):
    bsz = xp.shape[0]
    bm = BM_CONV
    while bsz % bm or bm < 8:
        bm //= 2
        if bm < 8:
            raise ValueError("batch must be a multiple of 8")
    return pl.pallas_call(
        _conv_stack_kernel,
        out_shape=jax.ShapeDtypeStruct((bsz, C2, POOL_HW), jnp.bfloat16),
        grid=(bsz // bm,),
        in_specs=[
            pl.BlockSpec((bm, C0P, XS), lambda i: (i, 0, 0)),
            pl.BlockSpec((C1, K1P), lambda i: (0, 0)),
            pl.BlockSpec((C1, 1), lambda i: (0, 0)),
            pl.BlockSpec((C2, K2), lambda i: (0, 0)),
            pl.BlockSpec((C2, 1), lambda i: (0, 0)),
            pl.BlockSpec((1, CW), lambda i: (0, 0)),
            pl.BlockSpec((IMGW, 2 * POOL_HW), lambda i: (0, 0)),
        ],
        out_specs=pl.BlockSpec((bm, C2, POOL_HW), lambda i: (i, 0, 0)),
        scratch_shapes=[
            pltpu.VMEM((K1P, CW), jnp.bfloat16),
            pltpu.VMEM((C1, BM_CONV * XS), jnp.bfloat16),
            pltpu.VMEM((K2, CW), jnp.bfloat16),
            pltpu.VMEM((C2 * BM_CONV, IMGW), jnp.bfloat16),
        ],
        compiler_params=pltpu.CompilerParams(
            dimension_semantics=("parallel",),
            vmem_limit_bytes=VMEM_LIMIT,
        ),
    )(xp, w1t, b1t, w2t, b2t, mask, gsel)


def _mlp_kernel(x_ref, w1_ref, b1_ref, w2_ref, b2_ref, o_ref):
    h = jnp.dot(x_ref[...], w1_ref[...], preferred_element_type=jnp.float32)
    h = jnp.maximum(h + b1_ref[...], 0.0).astype(jnp.bfloat16)
    o_ref[...] = jnp.dot(h, w2_ref[...], preferred_element_type=jnp.float32) + b2_ref[...]


def _mlp(flat, fw1, fb1, fw2, fb2):
    bsz = flat.shape[0]
    bm = BM_MLP if bsz % BM_MLP == 0 else bsz
    return pl.pallas_call(
        _mlp_kernel,
        out_shape=jax.ShapeDtypeStruct((bsz, FC2_OUT), jnp.float32),
        grid=(bsz // bm,),
        in_specs=[
            pl.BlockSpec((bm, FC1_IN), lambda i: (i, 0)),
            pl.BlockSpec((FC1_IN, FC1_OUT), lambda i: (0, 0)),
            pl.BlockSpec((1, FC1_OUT), lambda i: (0, 0)),
            pl.BlockSpec((FC1_OUT, FC2_OUT), lambda i: (0, 0)),
            pl.BlockSpec((1, FC2_OUT), lambda i: (0, 0)),
        ],
        out_specs=pl.BlockSpec((bm, FC2_OUT), lambda i: (i, 0)),
        compiler_params=pltpu.CompilerParams(
            dimension_semantics=("parallel",),
            vmem_limit_bytes=VMEM_LIMIT,
        ),
    )(flat, fw1, fb1, fw2, fb2)


def kernel(x, w1k, b1, w2k, b2, fw1, fb1, fw2, fb2):
    bsz = x.shape[0]
    xr = jnp.pad(x, ((0, 0), (0, 0), (0, 0), (0, WROW - W))).reshape(bsz, C_IN, IMGW)
    xp = jnp.pad(xr, ((0, 0), (0, C0P - C_IN), (HALO, HALO))).astype(jnp.bfloat16)
    w1t = jnp.pad(w1k.reshape(9, C_IN, C1),
                  ((0, 0), (0, C0P - C_IN), (0, 0))).reshape(K1P, C1)
    w1t = w1t.T.astype(jnp.bfloat16)
    w2t = w2k.T.astype(jnp.bfloat16)
    mask = jnp.asarray(_MASK_NP, jnp.bfloat16)[None, :]
    gsel = jnp.asarray(_G_NP, jnp.bfloat16)
    pooled = _conv_stack(xp, w1t, b1.T, w2t, b2.T, mask, gsel)
    flat = pooled.reshape(bsz, FC1_IN)
    return _mlp(flat, fw1.astype(jnp.bfloat16), fb1,
                fw2.astype(jnp.bfloat16), fb2)
